# Initial kernel scaffold; baseline (speedup 1.0000x reference)
import jax
import jax.numpy as jnp
from jax.experimental import pallas as pl


def kernel(x, edge_index, W1, b1, W2, b2):
    raise NotImplementedError("write your pallas kernel here")


# speedup vs baseline: 4.6557x; 4.6557x over previous
import functools

import jax
import jax.numpy as jnp
from jax import lax
from jax.experimental import pallas as pl
from jax.experimental.pallas import tpu as pltpu
from jax.experimental.pallas import tpu_sc as plsc

N = 10000
D = 256
E = 160000
HID = 256
OUT = 256
H = 128
CH = 128
NCHUNK = E // CH
NSUB = 16
NFULL = N // CH
TAIL = N - NFULL * CH
BLK = 1000


def _mlp1_body(x_ref, w1_ref, b1_ref, t_ref):
    xb = x_ref[...]
    a = jnp.dot(xb, w1_ref[:D, :], preferred_element_type=jnp.float32,
                precision=lax.Precision.HIGHEST) + b1_ref[...]
    b = jnp.dot(xb, w1_ref[D:, :], preferred_element_type=jnp.float32,
                precision=lax.Precision.HIGHEST)
    t_ref[0] = a[:, :H]
    t_ref[1] = a[:, H:]
    t_ref[2] = b[:, :H]
    t_ref[3] = b[:, H:]


def _mlp2_body(t_ref, s0_ref, s1_ref, c0_ref, c1_ref, w2_ref, b2_ref, o_ref):
    z0 = jnp.maximum(t_ref[0] + t_ref[2], 0.0)
    z1 = jnp.maximum(t_ref[1] + t_ref[3], 0.0)
    s0 = s0_ref[...] + z0
    s1 = s1_ref[...] + z1
    acc = jnp.dot(s0, w2_ref[:H, :], preferred_element_type=jnp.float32,
                  precision=lax.Precision.HIGHEST)
    acc = acc + jnp.dot(s1, w2_ref[H:, :], preferred_element_type=jnp.float32,
                        precision=lax.Precision.HIGHEST)
    cnt = c0_ref[:, 0:1] + c1_ref[:, 0:1] + 1.0
    o_ref[...] = acc / cnt + b2_ref[...]


def _edge_body(t_hbm, src_hbm, dst_hbm, s0_out, s1_out,
               sidx, didx, aidx, bidx, acc_a, acc_b, s_sp):
    cid = lax.axis_index("c")
    sid = lax.axis_index("s")
    zero16 = jnp.zeros((16,), jnp.float32)

    def zrow(r, carry):
        for j in range(8):
            acc_a[r, pl.ds(j * 16, 16)] = zero16
        return carry
    lax.fori_loop(0, CH, zrow, 0)

    for k in range(5):
        ck = sid + k * NSUB

        @pl.when(ck < NFULL)
        def _():
            pltpu.sync_copy(acc_a, s_sp.at[pl.ds(ck * CH, CH), :])

    @pl.when(sid == NSUB - 1)
    def _():
        pltpu.sync_copy(acc_a.at[pl.ds(0, TAIL), :],
                        s_sp.at[pl.ds(NFULL * CH, TAIL), :])

    plsc.subcore_barrier()

    aoff = jnp.full((16,), cid * N, jnp.int32)
    boff = jnp.full((16,), 2 * N + cid * N, jnp.int32)

    nch = (NCHUNK - sid + NSUB - 1) // NSUB

    def chunk_body(i, carry):
        ebase = (sid + i * NSUB) * CH
        pltpu.sync_copy(src_hbm.at[pl.ds(ebase, CH)], sidx)
        pltpu.sync_copy(dst_hbm.at[pl.ds(ebase, CH)], didx)
        for j in range(8):
            sl = pl.ds(j * 16, 16)
            aidx[sl] = sidx[sl] + aoff
            bidx[sl] = didx[sl] + boff
        pltpu.sync_copy(t_hbm.at[aidx], acc_a)
        pltpu.sync_copy(t_hbm.at[bidx], acc_b)

        def relu_row(r, c2):
            for j in range(8):
                sl = pl.ds(j * 16, 16)
                acc_a[r, sl] = jnp.maximum(acc_a[r, sl] + acc_b[r, sl], 0.0)
            return c2
        lax.fori_loop(0, CH, relu_row, 0)

        pltpu.sync_copy(acc_a, s_sp.at[didx], add=True)
        return carry
    lax.fori_loop(0, nch, chunk_body, 0)

    plsc.subcore_barrier()

    for k in range(5):
        ck = sid + k * NSUB
        r0 = ck * CH

        @pl.when(jnp.logical_and(ck < NFULL, cid == 0))
        def _():
            pltpu.sync_copy(s_sp.at[pl.ds(r0, CH), :], acc_a)
            pltpu.sync_copy(acc_a, s0_out.at[pl.ds(r0, CH), :])

        @pl.when(jnp.logical_and(ck < NFULL, cid == 1))
        def _():
            pltpu.sync_copy(s_sp.at[pl.ds(r0, CH), :], acc_a)
            pltpu.sync_copy(acc_a, s1_out.at[pl.ds(r0, CH), :])

    rt = NFULL * CH

    @pl.when(jnp.logical_and(sid == NSUB - 1, cid == 0))
    def _():
        pltpu.sync_copy(s_sp.at[pl.ds(rt, TAIL), :], acc_a.at[pl.ds(0, TAIL), :])
        pltpu.sync_copy(acc_a.at[pl.ds(0, TAIL), :], s0_out.at[pl.ds(rt, TAIL), :])

    @pl.when(jnp.logical_and(sid == NSUB - 1, cid == 1))
    def _():
        pltpu.sync_copy(s_sp.at[pl.ds(rt, TAIL), :], acc_a.at[pl.ds(0, TAIL), :])
        pltpu.sync_copy(acc_a.at[pl.ds(0, TAIL), :], s1_out.at[pl.ds(rt, TAIL), :])


def _count_body(dst_hbm, c0_out, c1_out, didx, obuf, cnt_sp):
    cid = lax.axis_index("c")
    sid = lax.axis_index("s")
    zero16 = jnp.zeros((16,), jnp.float32)
    one16 = jnp.ones((16,), jnp.float32)

    def zrow(r, carry):
        for j in range(8):
            obuf[r, pl.ds(j * 16, 16)] = zero16
        return carry
    lax.fori_loop(0, CH, zrow, 0)

    for k in range(5):
        ck = sid + k * NSUB

        @pl.when(ck < NFULL)
        def _():
            pltpu.sync_copy(obuf, cnt_sp.at[pl.ds(ck * CH, CH), :])

    @pl.when(sid == NSUB - 1)
    def _():
        pltpu.sync_copy(obuf.at[pl.ds(0, TAIL), :],
                        cnt_sp.at[pl.ds(NFULL * CH, TAIL), :])

    def orow(r, carry):
        for j in range(8):
            obuf[r, pl.ds(j * 16, 16)] = one16
        return carry
    lax.fori_loop(0, CH, orow, 0)

    plsc.subcore_barrier()

    wid = sid * 2 + cid
    nch = (NCHUNK - wid + 2 * NSUB - 1) // (2 * NSUB)

    def chunk_body(i, carry):
        ebase = (wid + i * 2 * NSUB) * CH
        pltpu.sync_copy(dst_hbm.at[pl.ds(ebase, CH)], didx)
        pltpu.sync_copy(obuf, cnt_sp.at[didx], add=True)
        return carry
    lax.fori_loop(0, nch, chunk_body, 0)

    plsc.subcore_barrier()

    for k in range(5):
        ck = sid + k * NSUB
        r0 = ck * CH

        @pl.when(jnp.logical_and(ck < NFULL, cid == 0))
        def _():
            pltpu.sync_copy(cnt_sp.at[pl.ds(r0, CH), :], obuf)
            pltpu.sync_copy(obuf, c0_out.at[pl.ds(r0, CH), :])

        @pl.when(jnp.logical_and(ck < NFULL, cid == 1))
        def _():
            pltpu.sync_copy(cnt_sp.at[pl.ds(r0, CH), :], obuf)
            pltpu.sync_copy(obuf, c1_out.at[pl.ds(r0, CH), :])

    rt = NFULL * CH

    @pl.when(jnp.logical_and(sid == NSUB - 1, cid == 0))
    def _():
        pltpu.sync_copy(cnt_sp.at[pl.ds(rt, TAIL), :], obuf.at[pl.ds(0, TAIL), :])
        pltpu.sync_copy(obuf.at[pl.ds(0, TAIL), :], c0_out.at[pl.ds(rt, TAIL), :])

    @pl.when(jnp.logical_and(sid == NSUB - 1, cid == 1))
    def _():
        pltpu.sync_copy(cnt_sp.at[pl.ds(rt, TAIL), :], obuf.at[pl.ds(0, TAIL), :])
        pltpu.sync_copy(obuf.at[pl.ds(0, TAIL), :], c1_out.at[pl.ds(rt, TAIL), :])


@functools.cache
def _edge_kernel():
    return functools.partial(
        pl.kernel,
        out_type=(
            jax.ShapeDtypeStruct((N, H), jnp.float32),
            jax.ShapeDtypeStruct((N, H), jnp.float32),
        ),
        mesh=plsc.VectorSubcoreMesh(core_axis_name="c", subcore_axis_name="s"),
        scratch_types=[
            pltpu.VMEM((CH,), jnp.int32),
            pltpu.VMEM((CH,), jnp.int32),
            pltpu.VMEM((CH,), jnp.int32),
            pltpu.VMEM((CH,), jnp.int32),
            pltpu.VMEM((CH, H), jnp.float32),
            pltpu.VMEM((CH, H), jnp.float32),
            pltpu.VMEM_SHARED((N, H), jnp.float32),
        ],
    )(_edge_body)


@functools.cache
def _count_kernel():
    return functools.partial(
        pl.kernel,
        out_type=(
            jax.ShapeDtypeStruct((N, H), jnp.float32),
            jax.ShapeDtypeStruct((N, H), jnp.float32),
        ),
        mesh=plsc.VectorSubcoreMesh(core_axis_name="c", subcore_axis_name="s"),
        scratch_types=[
            pltpu.VMEM((CH,), jnp.int32),
            pltpu.VMEM((CH, H), jnp.float32),
            pltpu.VMEM_SHARED((N, H), jnp.float32),
        ],
    )(_count_body)


def kernel(x, edge_index, W1, b1, W2, b2):
    src = edge_index[0]
    dst = edge_index[1]
    b1_2d = b1.reshape(1, HID)
    b2_2d = b2.reshape(1, OUT)

    t = pl.pallas_call(
        _mlp1_body,
        grid=(N // BLK,),
        in_specs=[
            pl.BlockSpec((BLK, D), lambda i: (i, 0)),
            pl.BlockSpec((2 * D, HID), lambda i: (0, 0)),
            pl.BlockSpec((1, HID), lambda i: (0, 0)),
        ],
        out_specs=pl.BlockSpec((4, BLK, H), lambda i: (0, i, 0)),
        out_shape=jax.ShapeDtypeStruct((4, N, H), jnp.float32),
    )(x, W1, b1_2d)

    s0, s1 = _edge_kernel()(t.reshape(4 * N, H), src, dst)
    c0, c1 = _count_kernel()(dst)

    out = pl.pallas_call(
        _mlp2_body,
        grid=(N // BLK,),
        in_specs=[
            pl.BlockSpec((4, BLK, H), lambda i: (0, i, 0)),
            pl.BlockSpec((BLK, H), lambda i: (i, 0)),
            pl.BlockSpec((BLK, H), lambda i: (i, 0)),
            pl.BlockSpec((BLK, H), lambda i: (i, 0)),
            pl.BlockSpec((BLK, H), lambda i: (i, 0)),
            pl.BlockSpec((HID, OUT), lambda i: (0, 0)),
            pl.BlockSpec((1, OUT), lambda i: (0, 0)),
        ],
        out_specs=pl.BlockSpec((BLK, OUT), lambda i: (i, 0)),
        out_shape=jax.ShapeDtypeStruct((N, OUT), jnp.float32),
    )(t, s0, s1, c0, c1, W2, b2_2d)
    return out


# speedup vs baseline: 7.8883x; 1.6943x over previous
import functools

import jax
import jax.numpy as jnp
from jax import lax
from jax.experimental import pallas as pl
from jax.experimental.pallas import tpu as pltpu
from jax.experimental.pallas import tpu_sc as plsc

N = 10000
D = 256
E = 160000
HID = 256
OUT = 256
H = 128
CH = 128
NCHUNK = E // CH
NSUB = 16
NFULL = N // CH
TAIL = N - NFULL * CH
NITER = 28
BLK = 1000


def _mlp1_body(x_ref, w1_ref, b1_ref, t_ref):
    xb = x_ref[...]
    a = jnp.dot(xb, w1_ref[:D, :], preferred_element_type=jnp.float32,
                precision=lax.Precision.HIGHEST) + b1_ref[...]
    b = jnp.dot(xb, w1_ref[D:, :], preferred_element_type=jnp.float32,
                precision=lax.Precision.HIGHEST)
    t_ref[0] = a[:, :H]
    t_ref[1] = a[:, H:]
    t_ref[2] = b[:, :H]
    t_ref[3] = b[:, H:]


def _mlp2_body(t_ref, s0_ref, s1_ref, c0_ref, c1_ref, w2_ref, b2_ref, o_ref):
    z0 = jnp.maximum(t_ref[0] + t_ref[2], 0.0)
    z1 = jnp.maximum(t_ref[1] + t_ref[3], 0.0)
    s0 = s0_ref[...] + z0
    s1 = s1_ref[...] + z1
    acc = jnp.dot(s0, w2_ref[:H, :], preferred_element_type=jnp.float32,
                  precision=lax.Precision.HIGHEST)
    acc = acc + jnp.dot(s1, w2_ref[H:, :], preferred_element_type=jnp.float32,
                        precision=lax.Precision.HIGHEST)
    cnt = c0_ref[:, 0:1] + c1_ref[:, 0:1] + 1.0
    o_ref[...] = acc / cnt + b2_ref[...]


def _edge_body(t_hbm, src_hbm, dst_hbm, s0_out, s1_out,
               a0, a1, a2, b0, b1, b2, d0, d1, d2,
               acc0, acc1, acc2, ga0, ga1, ga2, gb0, gb1, gb2,
               sc0, sc1, sc2, s_sp):
    cid = lax.axis_index("c")
    sid = lax.axis_index("s")
    aidx = [a0, a1, a2]
    bidx = [b0, b1, b2]
    didx = [d0, d1, d2]
    acc = [acc0, acc1, acc2]
    ga = [ga0, ga1, ga2]
    gb = [gb0, gb1, gb2]
    sc = [sc0, sc1, sc2]
    zero16 = jnp.zeros((16,), jnp.float32)
    acc_a = acc0

    def zrow(r, carry):
        for j in range(8):
            acc_a[r, pl.ds(j * 16, 16)] = zero16
        return carry
    lax.fori_loop(0, CH, zrow, 0)

    for k in range(5):
        ck = sid + k * NSUB

        @pl.when(ck < NFULL)
        def _():
            pltpu.sync_copy(acc_a, s_sp.at[pl.ds(ck * CH, CH), :])

    @pl.when(sid == NSUB - 1)
    def _():
        pltpu.sync_copy(acc_a.at[pl.ds(0, TAIL), :],
                        s_sp.at[pl.ds(NFULL * CH, TAIL), :])

    plsc.subcore_barrier()

    aoff = jnp.full((16,), cid * N, jnp.int32)
    boff = jnp.full((16,), 2 * N + cid * N, jnp.int32)

    nch = (NCHUNK - sid + NSUB - 1) // NSUB

    def valid(j):
        return jnp.logical_and(j >= 0, j < nch)

    def stage_idx_ga(j, p):
        @pl.when(valid(j))
        def _():
            ebase = (sid + j * NSUB) * CH
            pltpu.sync_copy(src_hbm.at[pl.ds(ebase, CH)], aidx[p])
            pltpu.sync_copy(dst_hbm.at[pl.ds(ebase, CH)], didx[p])
            for jj in range(8):
                sl = pl.ds(jj * 16, 16)
                aidx[p][sl] = aidx[p][sl] + aoff
                bidx[p][sl] = didx[p][sl] + boff
            pltpu.make_async_copy(t_hbm.at[aidx[p]], acc[p], ga[p]).start()

    def stage_gb(j, p):
        @pl.when(valid(j))
        def _():
            pltpu.make_async_copy(t_hbm.at[aidx[p]], acc[p], ga[p]).wait()
            pltpu.make_async_copy(t_hbm.at[bidx[p]], acc[p], gb[p]).start(add=True)

    def stage_relu_scatter(j, p):
        @pl.when(valid(j))
        def _():
            pltpu.make_async_copy(t_hbm.at[bidx[p]], acc[p], gb[p]).wait()
            acc_p = acc[p]

            def relu_rows(r, c2):
                for half in range(2):
                    rr = r * 2 + half
                    for jj in range(8):
                        sl = pl.ds(jj * 16, 16)
                        acc_p[rr, sl] = jnp.maximum(acc_p[rr, sl], 0.0)
                return c2
            lax.fori_loop(0, CH // 2, relu_rows, 0)
            pltpu.make_async_copy(acc[p], s_sp.at[didx[p]], sc[p]).start(add=True)

    def stage_wait_scatter(j, p):
        @pl.when(valid(j))
        def _():
            pltpu.make_async_copy(acc[p], s_sp.at[didx[p]], sc[p]).wait()

    def outer(g, carry):
        for b in range(3):
            i = g * 3 + b
            stage_wait_scatter(i - 3, b)
            stage_idx_ga(i, b)
            stage_gb(i - 1, (b - 1) % 3)
            stage_relu_scatter(i - 2, (b - 2) % 3)
        return carry
    lax.fori_loop(0, NITER, outer, 0)

    plsc.subcore_barrier()

    for k in range(5):
        ck = sid + k * NSUB
        r0 = ck * CH

        @pl.when(jnp.logical_and(ck < NFULL, cid == 0))
        def _():
            pltpu.sync_copy(s_sp.at[pl.ds(r0, CH), :], acc_a)
            pltpu.sync_copy(acc_a, s0_out.at[pl.ds(r0, CH), :])

        @pl.when(jnp.logical_and(ck < NFULL, cid == 1))
        def _():
            pltpu.sync_copy(s_sp.at[pl.ds(r0, CH), :], acc_a)
            pltpu.sync_copy(acc_a, s1_out.at[pl.ds(r0, CH), :])

    rt = NFULL * CH

    @pl.when(jnp.logical_and(sid == NSUB - 1, cid == 0))
    def _():
        pltpu.sync_copy(s_sp.at[pl.ds(rt, TAIL), :], acc_a.at[pl.ds(0, TAIL), :])
        pltpu.sync_copy(acc_a.at[pl.ds(0, TAIL), :], s0_out.at[pl.ds(rt, TAIL), :])

    @pl.when(jnp.logical_and(sid == NSUB - 1, cid == 1))
    def _():
        pltpu.sync_copy(s_sp.at[pl.ds(rt, TAIL), :], acc_a.at[pl.ds(0, TAIL), :])
        pltpu.sync_copy(acc_a.at[pl.ds(0, TAIL), :], s1_out.at[pl.ds(rt, TAIL), :])


def _count_body(dst_hbm, c0_out, c1_out, didx, obuf, cnt_sp):
    cid = lax.axis_index("c")
    sid = lax.axis_index("s")
    zero16 = jnp.zeros((16,), jnp.float32)
    one16 = jnp.ones((16,), jnp.float32)

    def zrow(r, carry):
        for j in range(8):
            obuf[r, pl.ds(j * 16, 16)] = zero16
        return carry
    lax.fori_loop(0, CH, zrow, 0)

    for k in range(5):
        ck = sid + k * NSUB

        @pl.when(ck < NFULL)
        def _():
            pltpu.sync_copy(obuf, cnt_sp.at[pl.ds(ck * CH, CH), :])

    @pl.when(sid == NSUB - 1)
    def _():
        pltpu.sync_copy(obuf.at[pl.ds(0, TAIL), :],
                        cnt_sp.at[pl.ds(NFULL * CH, TAIL), :])

    def orow(r, carry):
        for j in range(8):
            obuf[r, pl.ds(j * 16, 16)] = one16
        return carry
    lax.fori_loop(0, CH, orow, 0)

    plsc.subcore_barrier()

    wid = sid * 2 + cid
    nch = (NCHUNK - wid + 2 * NSUB - 1) // (2 * NSUB)

    def chunk_body(i, carry):
        ebase = (wid + i * 2 * NSUB) * CH
        pltpu.sync_copy(dst_hbm.at[pl.ds(ebase, CH)], didx)
        pltpu.sync_copy(obuf, cnt_sp.at[didx], add=True)
        return carry
    lax.fori_loop(0, nch, chunk_body, 0)

    plsc.subcore_barrier()

    for k in range(5):
        ck = sid + k * NSUB
        r0 = ck * CH

        @pl.when(jnp.logical_and(ck < NFULL, cid == 0))
        def _():
            pltpu.sync_copy(cnt_sp.at[pl.ds(r0, CH), :], obuf)
            pltpu.sync_copy(obuf, c0_out.at[pl.ds(r0, CH), :])

        @pl.when(jnp.logical_and(ck < NFULL, cid == 1))
        def _():
            pltpu.sync_copy(cnt_sp.at[pl.ds(r0, CH), :], obuf)
            pltpu.sync_copy(obuf, c1_out.at[pl.ds(r0, CH), :])

    rt = NFULL * CH

    @pl.when(jnp.logical_and(sid == NSUB - 1, cid == 0))
    def _():
        pltpu.sync_copy(cnt_sp.at[pl.ds(rt, TAIL), :], obuf.at[pl.ds(0, TAIL), :])
        pltpu.sync_copy(obuf.at[pl.ds(0, TAIL), :], c0_out.at[pl.ds(rt, TAIL), :])

    @pl.when(jnp.logical_and(sid == NSUB - 1, cid == 1))
    def _():
        pltpu.sync_copy(cnt_sp.at[pl.ds(rt, TAIL), :], obuf.at[pl.ds(0, TAIL), :])
        pltpu.sync_copy(obuf.at[pl.ds(0, TAIL), :], c1_out.at[pl.ds(rt, TAIL), :])


@functools.cache
def _edge_kernel():
    return functools.partial(
        pl.kernel,
        out_type=(
            jax.ShapeDtypeStruct((N, H), jnp.float32),
            jax.ShapeDtypeStruct((N, H), jnp.float32),
        ),
        mesh=plsc.VectorSubcoreMesh(core_axis_name="c", subcore_axis_name="s"),
        scratch_types=(
            [pltpu.VMEM((CH,), jnp.int32)] * 9
            + [pltpu.VMEM((CH, H), jnp.float32)] * 3
            + [pltpu.SemaphoreType.DMA] * 9
            + [pltpu.VMEM_SHARED((N, H), jnp.float32)]
        ),
    )(_edge_body)


@functools.cache
def _count_kernel():
    return functools.partial(
        pl.kernel,
        out_type=(
            jax.ShapeDtypeStruct((N, H), jnp.float32),
            jax.ShapeDtypeStruct((N, H), jnp.float32),
        ),
        mesh=plsc.VectorSubcoreMesh(core_axis_name="c", subcore_axis_name="s"),
        scratch_types=[
            pltpu.VMEM((CH,), jnp.int32),
            pltpu.VMEM((CH, H), jnp.float32),
            pltpu.VMEM_SHARED((N, H), jnp.float32),
        ],
    )(_count_body)


def kernel(x, edge_index, W1, b1, W2, b2):
    src = edge_index[0]
    dst = edge_index[1]
    b1_2d = b1.reshape(1, HID)
    b2_2d = b2.reshape(1, OUT)

    t = pl.pallas_call(
        _mlp1_body,
        grid=(N // BLK,),
        in_specs=[
            pl.BlockSpec((BLK, D), lambda i: (i, 0)),
            pl.BlockSpec((2 * D, HID), lambda i: (0, 0)),
            pl.BlockSpec((1, HID), lambda i: (0, 0)),
        ],
        out_specs=pl.BlockSpec((4, BLK, H), lambda i: (0, i, 0)),
        out_shape=jax.ShapeDtypeStruct((4, N, H), jnp.float32),
    )(x, W1, b1_2d)

    s0, s1 = _edge_kernel()(t.reshape(4 * N, H), src, dst)
    c0, c1 = _count_kernel()(dst)

    out = pl.pallas_call(
        _mlp2_body,
        grid=(N // BLK,),
        in_specs=[
            pl.BlockSpec((4, BLK, H), lambda i: (0, i, 0)),
            pl.BlockSpec((BLK, H), lambda i: (i, 0)),
            pl.BlockSpec((BLK, H), lambda i: (i, 0)),
            pl.BlockSpec((BLK, H), lambda i: (i, 0)),
            pl.BlockSpec((BLK, H), lambda i: (i, 0)),
            pl.BlockSpec((HID, OUT), lambda i: (0, 0)),
            pl.BlockSpec((1, OUT), lambda i: (0, 0)),
        ],
        out_specs=pl.BlockSpec((BLK, OUT), lambda i: (i, 0)),
        out_shape=jax.ShapeDtypeStruct((N, OUT), jnp.float32),
    )(t, s0, s1, c0, c1, W2, b2_2d)
    return out


# speedup vs baseline: 8.5287x; 1.0812x over previous
import functools

import jax
import jax.numpy as jnp
from jax import lax
from jax.experimental import pallas as pl
from jax.experimental.pallas import tpu as pltpu
from jax.experimental.pallas import tpu_sc as plsc

N = 10000
D = 256
E = 160000
HID = 256
OUT = 256
H = 128
CH = 128
NCHUNK = E // CH
NSUB = 16
NFULL = N // CH
TAIL = N - NFULL * CH
NITER = 28
CNITER = 21
BLK = 1000


def _mlp1_body(x_ref, w1_ref, b1_ref, t_ref):
    xb = x_ref[...]
    a = jnp.dot(xb, w1_ref[:D, :], preferred_element_type=jnp.float32,
                precision=lax.Precision.DEFAULT) + b1_ref[...]
    b = jnp.dot(xb, w1_ref[D:, :], preferred_element_type=jnp.float32,
                precision=lax.Precision.DEFAULT)
    t_ref[0] = a[:, :H]
    t_ref[1] = a[:, H:]
    t_ref[2] = b[:, :H]
    t_ref[3] = b[:, H:]


def _mlp2_body(t_ref, s0_ref, s1_ref, c0_ref, c1_ref, w2_ref, b2_ref, o_ref):
    z0 = jnp.maximum(t_ref[0] + t_ref[2], 0.0)
    z1 = jnp.maximum(t_ref[1] + t_ref[3], 0.0)
    s0 = s0_ref[...] + z0
    s1 = s1_ref[...] + z1
    acc = jnp.dot(s0, w2_ref[:H, :], preferred_element_type=jnp.float32,
                  precision=lax.Precision.DEFAULT)
    acc = acc + jnp.dot(s1, w2_ref[H:, :], preferred_element_type=jnp.float32,
                        precision=lax.Precision.DEFAULT)
    cnt = c0_ref[:, 0:1] + c1_ref[:, 0:1] + 1.0
    o_ref[...] = acc / cnt + b2_ref[...]


def _edge_body(t_hbm, src_hbm, dst_hbm, s0_out, s1_out,
               a0, a1, a2, b0, b1, b2, d0, d1, d2,
               acc0, acc1, acc2, ga0, ga1, ga2, gb0, gb1, gb2,
               sc0, sc1, sc2, s_sp):
    cid = lax.axis_index("c")
    sid = lax.axis_index("s")
    aidx = [a0, a1, a2]
    bidx = [b0, b1, b2]
    didx = [d0, d1, d2]
    acc = [acc0, acc1, acc2]
    ga = [ga0, ga1, ga2]
    gb = [gb0, gb1, gb2]
    sc = [sc0, sc1, sc2]
    zero16 = jnp.zeros((16,), jnp.float32)
    acc_a = acc0

    def zrow(r, carry):
        for j in range(8):
            acc_a[r, pl.ds(j * 16, 16)] = zero16
        return carry
    lax.fori_loop(0, CH, zrow, 0)

    for k in range(5):
        ck = sid + k * NSUB

        @pl.when(ck < NFULL)
        def _():
            pltpu.sync_copy(acc_a, s_sp.at[pl.ds(ck * CH, CH), :])

    @pl.when(sid == NSUB - 1)
    def _():
        pltpu.sync_copy(acc_a.at[pl.ds(0, TAIL), :],
                        s_sp.at[pl.ds(NFULL * CH, TAIL), :])

    plsc.subcore_barrier()

    aoff = jnp.full((16,), cid * N, jnp.int32)
    boff = jnp.full((16,), 2 * N + cid * N, jnp.int32)

    nch = (NCHUNK - sid + NSUB - 1) // NSUB

    def valid(j):
        return jnp.logical_and(j >= 0, j < nch)

    def stage_idx_ga(j, p):
        @pl.when(valid(j))
        def _():
            ebase = (sid + j * NSUB) * CH
            pltpu.sync_copy(src_hbm.at[pl.ds(ebase, CH)], aidx[p])
            pltpu.sync_copy(dst_hbm.at[pl.ds(ebase, CH)], didx[p])
            for jj in range(8):
                sl = pl.ds(jj * 16, 16)
                aidx[p][sl] = aidx[p][sl] + aoff
                bidx[p][sl] = didx[p][sl] + boff
            pltpu.make_async_copy(t_hbm.at[aidx[p]], acc[p], ga[p]).start()

    def stage_gb(j, p):
        @pl.when(valid(j))
        def _():
            pltpu.make_async_copy(t_hbm.at[aidx[p]], acc[p], ga[p]).wait()
            pltpu.make_async_copy(t_hbm.at[bidx[p]], acc[p], gb[p]).start(add=True)

    def stage_relu_scatter(j, p):
        @pl.when(valid(j))
        def _():
            pltpu.make_async_copy(t_hbm.at[bidx[p]], acc[p], gb[p]).wait()
            acc_p = acc[p]

            def relu_rows(r, c2):
                for half in range(2):
                    rr = r * 2 + half
                    for jj in range(8):
                        sl = pl.ds(jj * 16, 16)
                        acc_p[rr, sl] = jnp.maximum(acc_p[rr, sl], 0.0)
                return c2
            lax.fori_loop(0, CH // 2, relu_rows, 0)
            pltpu.make_async_copy(acc[p], s_sp.at[didx[p]], sc[p]).start(add=True)

    def stage_wait_scatter(j, p):
        @pl.when(valid(j))
        def _():
            pltpu.make_async_copy(acc[p], s_sp.at[didx[p]], sc[p]).wait()

    def outer(g, carry):
        for b in range(3):
            i = g * 3 + b
            stage_wait_scatter(i - 3, b)
            stage_idx_ga(i, b)
            stage_gb(i - 1, (b - 1) % 3)
            stage_relu_scatter(i - 2, (b - 2) % 3)
        return carry
    lax.fori_loop(0, NITER, outer, 0)

    plsc.subcore_barrier()

    for k in range(5):
        ck = sid + k * NSUB
        r0 = ck * CH

        @pl.when(jnp.logical_and(ck < NFULL, cid == 0))
        def _():
            pltpu.sync_copy(s_sp.at[pl.ds(r0, CH), :], acc_a)
            pltpu.sync_copy(acc_a, s0_out.at[pl.ds(r0, CH), :])

        @pl.when(jnp.logical_and(ck < NFULL, cid == 1))
        def _():
            pltpu.sync_copy(s_sp.at[pl.ds(r0, CH), :], acc_a)
            pltpu.sync_copy(acc_a, s1_out.at[pl.ds(r0, CH), :])

    rt = NFULL * CH

    @pl.when(jnp.logical_and(sid == NSUB - 1, cid == 0))
    def _():
        pltpu.sync_copy(s_sp.at[pl.ds(rt, TAIL), :], acc_a.at[pl.ds(0, TAIL), :])
        pltpu.sync_copy(acc_a.at[pl.ds(0, TAIL), :], s0_out.at[pl.ds(rt, TAIL), :])

    @pl.when(jnp.logical_and(sid == NSUB - 1, cid == 1))
    def _():
        pltpu.sync_copy(s_sp.at[pl.ds(rt, TAIL), :], acc_a.at[pl.ds(0, TAIL), :])
        pltpu.sync_copy(acc_a.at[pl.ds(0, TAIL), :], s1_out.at[pl.ds(rt, TAIL), :])


def _count_body(dst_hbm, c0_out, c1_out, cd0, cd1, obuf, cs0, cs1, cnt_sp):
    didx = [cd0, cd1]
    csem = [cs0, cs1]
    cid = lax.axis_index("c")
    sid = lax.axis_index("s")
    zero16 = jnp.zeros((16,), jnp.float32)
    one16 = jnp.ones((16,), jnp.float32)

    def zrow(r, carry):
        for j in range(8):
            obuf[r, pl.ds(j * 16, 16)] = zero16
        return carry
    lax.fori_loop(0, CH, zrow, 0)

    for k in range(5):
        ck = sid + k * NSUB

        @pl.when(ck < NFULL)
        def _():
            pltpu.sync_copy(obuf, cnt_sp.at[pl.ds(ck * CH, CH), :])

    @pl.when(sid == NSUB - 1)
    def _():
        pltpu.sync_copy(obuf.at[pl.ds(0, TAIL), :],
                        cnt_sp.at[pl.ds(NFULL * CH, TAIL), :])

    def orow(r, carry):
        for j in range(8):
            obuf[r, pl.ds(j * 16, 16)] = one16
        return carry
    lax.fori_loop(0, CH, orow, 0)

    plsc.subcore_barrier()

    wid = sid * 2 + cid
    nch = (NCHUNK - wid + 2 * NSUB - 1) // (2 * NSUB)

    def cvalid(j):
        return jnp.logical_and(j >= 0, j < nch)

    def c_stage_idx_scatter(j, p):
        @pl.when(cvalid(j))
        def _():
            ebase = (wid + j * 2 * NSUB) * CH
            pltpu.sync_copy(dst_hbm.at[pl.ds(ebase, CH)], didx[p])
            pltpu.make_async_copy(obuf, cnt_sp.at[didx[p]], csem[p]).start(add=True)

    def c_stage_wait(j, p):
        @pl.when(cvalid(j))
        def _():
            pltpu.make_async_copy(obuf, cnt_sp.at[didx[p]], csem[p]).wait()

    def c_outer(g, carry):
        for b in range(2):
            i = g * 2 + b
            c_stage_wait(i - 2, b)
            c_stage_idx_scatter(i, b)
        return carry
    lax.fori_loop(0, CNITER, c_outer, 0)

    plsc.subcore_barrier()

    for k in range(5):
        ck = sid + k * NSUB
        r0 = ck * CH

        @pl.when(jnp.logical_and(ck < NFULL, cid == 0))
        def _():
            pltpu.sync_copy(cnt_sp.at[pl.ds(r0, CH), :], obuf)
            pltpu.sync_copy(obuf, c0_out.at[pl.ds(r0, CH), :])

        @pl.when(jnp.logical_and(ck < NFULL, cid == 1))
        def _():
            pltpu.sync_copy(cnt_sp.at[pl.ds(r0, CH), :], obuf)
            pltpu.sync_copy(obuf, c1_out.at[pl.ds(r0, CH), :])

    rt = NFULL * CH

    @pl.when(jnp.logical_and(sid == NSUB - 1, cid == 0))
    def _():
        pltpu.sync_copy(cnt_sp.at[pl.ds(rt, TAIL), :], obuf.at[pl.ds(0, TAIL), :])
        pltpu.sync_copy(obuf.at[pl.ds(0, TAIL), :], c0_out.at[pl.ds(rt, TAIL), :])

    @pl.when(jnp.logical_and(sid == NSUB - 1, cid == 1))
    def _():
        pltpu.sync_copy(cnt_sp.at[pl.ds(rt, TAIL), :], obuf.at[pl.ds(0, TAIL), :])
        pltpu.sync_copy(obuf.at[pl.ds(0, TAIL), :], c1_out.at[pl.ds(rt, TAIL), :])


@functools.cache
def _edge_kernel():
    return functools.partial(
        pl.kernel,
        out_type=(
            jax.ShapeDtypeStruct((N, H), jnp.float32),
            jax.ShapeDtypeStruct((N, H), jnp.float32),
        ),
        mesh=plsc.VectorSubcoreMesh(core_axis_name="c", subcore_axis_name="s"),
        scratch_types=(
            [pltpu.VMEM((CH,), jnp.int32)] * 9
            + [pltpu.VMEM((CH, H), jnp.float32)] * 3
            + [pltpu.SemaphoreType.DMA] * 9
            + [pltpu.VMEM_SHARED((N, H), jnp.float32)]
        ),
    )(_edge_body)


@functools.cache
def _count_kernel():
    return functools.partial(
        pl.kernel,
        out_type=(
            jax.ShapeDtypeStruct((N, H), jnp.float32),
            jax.ShapeDtypeStruct((N, H), jnp.float32),
        ),
        mesh=plsc.VectorSubcoreMesh(core_axis_name="c", subcore_axis_name="s"),
        scratch_types=[
            pltpu.VMEM((CH,), jnp.int32),
            pltpu.VMEM((CH,), jnp.int32),
            pltpu.VMEM((CH, H), jnp.float32),
            pltpu.SemaphoreType.DMA,
            pltpu.SemaphoreType.DMA,
            pltpu.VMEM_SHARED((N, H), jnp.float32),
        ],
    )(_count_body)


def kernel(x, edge_index, W1, b1, W2, b2):
    src = edge_index[0]
    dst = edge_index[1]
    b1_2d = b1.reshape(1, HID)
    b2_2d = b2.reshape(1, OUT)

    t = pl.pallas_call(
        _mlp1_body,
        grid=(N // BLK,),
        in_specs=[
            pl.BlockSpec((BLK, D), lambda i: (i, 0)),
            pl.BlockSpec((2 * D, HID), lambda i: (0, 0)),
            pl.BlockSpec((1, HID), lambda i: (0, 0)),
        ],
        out_specs=pl.BlockSpec((4, BLK, H), lambda i: (0, i, 0)),
        out_shape=jax.ShapeDtypeStruct((4, N, H), jnp.float32),
    )(x, W1, b1_2d)

    s0, s1 = _edge_kernel()(t.reshape(4 * N, H), src, dst)
    c0, c1 = _count_kernel()(dst)

    out = pl.pallas_call(
        _mlp2_body,
        grid=(N // BLK,),
        in_specs=[
            pl.BlockSpec((4, BLK, H), lambda i: (0, i, 0)),
            pl.BlockSpec((BLK, H), lambda i: (i, 0)),
            pl.BlockSpec((BLK, H), lambda i: (i, 0)),
            pl.BlockSpec((BLK, H), lambda i: (i, 0)),
            pl.BlockSpec((BLK, H), lambda i: (i, 0)),
            pl.BlockSpec((HID, OUT), lambda i: (0, 0)),
            pl.BlockSpec((1, OUT), lambda i: (0, 0)),
        ],
        out_specs=pl.BlockSpec((BLK, OUT), lambda i: (i, 0)),
        out_shape=jax.ShapeDtypeStruct((N, OUT), jnp.float32),
    )(t, s0, s1, c0, c1, W2, b2_2d)
    return out
